# baseline (speedup 1.0000x reference)
import functools
import math

import jax
import jax.numpy as jnp
from jax import lax
from jax.experimental import pallas as pl
from jax.experimental.pallas import tpu as pltpu
from jax.experimental.pallas import tpu_sc as plsc

_K = 8
_BS = 2


def _loss_body(pf_ref, x_ref, t_ref, ml_ref, idx_ref):
    _, h, w, d = t_ref.shape
    inv_n = 1.0 / (h * w * d)
    penalty = math.log(_K, 2) / (h * w)
    b = pl.program_id(0)

    for s in range(_BS):
        tb = t_ref[s]

        best_scaled = jnp.float32(jnp.inf)
        best_loss = jnp.float32(0.0)
        best_idx = jnp.int32(0)
        for k in range(_K):
            diff = x_ref[s, :, :, k * d:(k + 1) * d] - tb
            loss_k = jnp.sum(diff * diff) * inv_n + penalty
            scaled_k = loss_k * pf_ref[0, k]
            better = scaled_k < best_scaled
            best_scaled = jnp.where(better, scaled_k, best_scaled)
            best_loss = jnp.where(better, loss_k, best_loss)
            best_idx = jnp.where(better, jnp.int32(k), best_idx)

        ml_ref[s] = jnp.full((1, 128), best_loss, jnp.float32)
        g = b * _BS + s
        p = (lax.broadcasted_iota(jnp.int32, (_K, 128), 0) * 128
             + lax.broadcasted_iota(jnp.int32, (_K, 128), 1))
        idx_ref[s] = g * 8192 + (p >> 3) * 64 + best_idx * 8 + (p & 7)


def _make_sc_gather(n_rows, nj):
    info = plsc.get_sparse_core_info()
    nc, ns = info.num_cores, info.num_subcores
    mesh = plsc.VectorSubcoreMesh(core_axis_name="c", subcore_axis_name="s")

    @functools.partial(
        pl.kernel,
        mesh=mesh,
        out_type=jax.ShapeDtypeStruct((n_rows, 128), jnp.float32),
        scratch_types=[
            pltpu.VMEM((nj, 128), jnp.int32),
            pltpu.VMEM((128, 128), jnp.float32),
            pltpu.SemaphoreType.DMA,
        ],
    )
    def gather(table_hbm, idx_hbm, out_hbm, idx_v, rows_v, sem):
        wid = lax.axis_index("s") * nc + lax.axis_index("c")
        pltpu.sync_copy(idx_hbm.at[wid], idx_v)
        for j in range(nj):
            pltpu.async_copy(table_hbm.at[idx_v.at[j]], rows_v, sem).wait()
            pltpu.sync_copy(rows_v, out_hbm.at[pl.ds(wid * nj * 128 + j * 128, 128)])

    return gather


def kernel(x, target, pick_frequency):
    B, C, H, W = x.shape
    D = C // _K
    xt = jnp.transpose(x, (0, 2, 3, 1))
    tt = jnp.transpose(target, (0, 2, 3, 1))
    pf = pick_frequency.reshape(1, _K)

    ml, idx = pl.pallas_call(
        _loss_body,
        grid=(B // _BS,),
        in_specs=[
            pl.BlockSpec(memory_space=pltpu.SMEM),
            pl.BlockSpec((_BS, H, W, C), lambda b: (b, 0, 0, 0)),
            pl.BlockSpec((_BS, H, W, D), lambda b: (b, 0, 0, 0)),
        ],
        out_specs=[
            pl.BlockSpec((_BS, 1, 128), lambda b: (b, 0, 0)),
            pl.BlockSpec((_BS, _K, 128), lambda b: (b, 0, 0)),
        ],
        out_shape=[
            jax.ShapeDtypeStruct((B, 1, 128), jnp.float32),
            jax.ShapeDtypeStruct((B, _K, 128), jnp.int32),
        ],
        compiler_params=pltpu.CompilerParams(
            dimension_semantics=("parallel",),
        ),
    )(pf, xt, tt)

    table = (xt.reshape(B, H, W // 8, 8, _K, 128)
             .transpose(0, 1, 2, 4, 3, 5)
             .reshape(B * H * W * _K, 128))

    sel_rows = _make_sc_gather(B * H * W, _K)(table, idx)

    sel = sel_rows.reshape(B, H, W, D)
    selected = jnp.transpose(sel, (0, 3, 1, 2))
    min_loss = ml[:, 0, 0]
    return selected, min_loss


# speedup vs baseline: 1.0617x; 1.0617x over previous
import functools
import math

import jax
import jax.numpy as jnp
from jax import lax
from jax.experimental import pallas as pl
from jax.experimental.pallas import tpu as pltpu
from jax.experimental.pallas import tpu_sc as plsc

_K = 8
_BS = 2


def _loss_body(pf_ref, x_ref, t_ref, ml_ref, idx_ref):
    _, h, w, d = t_ref.shape
    inv_n = 1.0 / (h * w * d)
    penalty = math.log(_K, 2) / (h * w)
    b = pl.program_id(0)

    for s in range(_BS):
        tb = t_ref[s]

        best_scaled = jnp.float32(jnp.inf)
        best_loss = jnp.float32(0.0)
        best_idx = jnp.int32(0)
        for k in range(_K):
            diff = x_ref[s, :, :, k * d:(k + 1) * d] - tb
            loss_k = jnp.sum(diff * diff) * inv_n + penalty
            scaled_k = loss_k * pf_ref[0, k]
            better = scaled_k < best_scaled
            best_scaled = jnp.where(better, scaled_k, best_scaled)
            best_loss = jnp.where(better, loss_k, best_loss)
            best_idx = jnp.where(better, jnp.int32(k), best_idx)

        ml_ref[s] = jnp.full((1, 128), best_loss, jnp.float32)
        g = b * _BS + s
        p = (lax.broadcasted_iota(jnp.int32, (_K, 128), 0) * 128
             + lax.broadcasted_iota(jnp.int32, (_K, 128), 1))
        idx_ref[s] = g * 8192 + (p >> 3) * 64 + best_idx * 8 + (p & 7)


def _make_sc_gather(n_rows, nj):
    info = plsc.get_sparse_core_info()
    nc, ns = info.num_cores, info.num_subcores
    mesh = plsc.VectorSubcoreMesh(core_axis_name="c", subcore_axis_name="s")

    @functools.partial(
        pl.kernel,
        mesh=mesh,
        out_type=jax.ShapeDtypeStruct((n_rows, 128), jnp.float32),
        scratch_types=[
            pltpu.VMEM((nj, 128), jnp.int32),
            pltpu.VMEM((2, 128, 128), jnp.float32),
            pltpu.SemaphoreType.DMA,
            pltpu.SemaphoreType.DMA,
            pltpu.SemaphoreType.DMA,
            pltpu.SemaphoreType.DMA,
        ],
    )
    def gather(table_hbm, idx_hbm, out_hbm, idx_v, rows_v, g0, g1, o0, o1):
        wid = lax.axis_index("s") * nc + lax.axis_index("c")
        base = wid * nj * 128
        gsem = (g0, g1)
        osem = (o0, o1)
        pltpu.sync_copy(idx_hbm.at[wid], idx_v)
        gathers = [None, None]
        writes = [None, None]
        for j in range(nj):
            m = j % 2
            if writes[m] is not None:
                writes[m].wait()
                writes[m] = None
            gathers[m] = pltpu.async_copy(
                table_hbm.at[idx_v.at[j]], rows_v.at[m], gsem[m])
            if j > 0:
                pm = 1 - m
                gathers[pm].wait()
                writes[pm] = pltpu.async_copy(
                    rows_v.at[pm], out_hbm.at[pl.ds(base + (j - 1) * 128, 128)],
                    osem[pm])
        m = (nj - 1) % 2
        pm = 1 - m
        gathers[m].wait()
        pltpu.sync_copy(rows_v.at[m], out_hbm.at[pl.ds(base + (nj - 1) * 128, 128)])
        if writes[pm] is not None:
            writes[pm].wait()

    return gather


def kernel(x, target, pick_frequency):
    B, C, H, W = x.shape
    D = C // _K
    xt = jnp.transpose(x, (0, 2, 3, 1))
    tt = jnp.transpose(target, (0, 2, 3, 1))
    pf = pick_frequency.reshape(1, _K)

    ml, idx = pl.pallas_call(
        _loss_body,
        grid=(B // _BS,),
        in_specs=[
            pl.BlockSpec(memory_space=pltpu.SMEM),
            pl.BlockSpec((_BS, H, W, C), lambda b: (b, 0, 0, 0)),
            pl.BlockSpec((_BS, H, W, D), lambda b: (b, 0, 0, 0)),
        ],
        out_specs=[
            pl.BlockSpec((_BS, 1, 128), lambda b: (b, 0, 0)),
            pl.BlockSpec((_BS, _K, 128), lambda b: (b, 0, 0)),
        ],
        out_shape=[
            jax.ShapeDtypeStruct((B, 1, 128), jnp.float32),
            jax.ShapeDtypeStruct((B, _K, 128), jnp.int32),
        ],
        compiler_params=pltpu.CompilerParams(
            dimension_semantics=("parallel",),
        ),
    )(pf, xt, tt)

    table = (xt.reshape(B, H, W // 8, 8, _K, 128)
             .transpose(0, 1, 2, 4, 3, 5)
             .reshape(B * H * W * _K, 128))

    sel_rows = _make_sc_gather(B * H * W, _K)(table, idx)

    sel = sel_rows.reshape(B, H, W, D)
    selected = jnp.transpose(sel, (0, 3, 1, 2))
    min_loss = ml[:, 0, 0]
    return selected, min_loss


# speedup vs baseline: 1.4968x; 1.4099x over previous
import math

import jax
import jax.numpy as jnp
from jax.experimental import pallas as pl
from jax.experimental.pallas import tpu as pltpu

_K = 8
_BS = 2


def _body(pf_ref, x_ref, t_ref, sel_ref, ml_ref):
    _, h, w, d = t_ref.shape
    inv_n = 1.0 / (h * w * d)
    penalty = math.log(_K, 2) / (h * w)

    for s in range(_BS):
        tb = t_ref[s]

        best_scaled = jnp.float32(jnp.inf)
        best_loss = jnp.float32(0.0)
        best_idx = jnp.int32(0)
        for k in range(_K):
            chunk = x_ref[s, :, :, k * d:(k + 1) * d]
            diff = chunk - tb
            loss_k = jnp.sum(diff * diff) * inv_n + penalty
            scaled_k = loss_k * pf_ref[0, k]
            better = scaled_k < best_scaled
            best_scaled = jnp.where(better, scaled_k, best_scaled)
            best_loss = jnp.where(better, loss_k, best_loss)
            best_idx = jnp.where(better, jnp.int32(k), best_idx)

        ml_ref[s] = jnp.full((1, 128), best_loss, jnp.float32)
        for k in range(_K):
            @pl.when(best_idx == k)
            def _():
                sel_ref[s] = x_ref[s, :, :, k * d:(k + 1) * d]


def kernel(x, target, pick_frequency):
    B, C, H, W = x.shape
    D = C // _K
    xt = jnp.transpose(x, (0, 2, 3, 1))
    tt = jnp.transpose(target, (0, 2, 3, 1))
    pf = pick_frequency.reshape(1, _K)

    sel, ml = pl.pallas_call(
        _body,
        grid=(B // _BS,),
        in_specs=[
            pl.BlockSpec(memory_space=pltpu.SMEM),
            pl.BlockSpec((_BS, H, W, C), lambda b: (b, 0, 0, 0)),
            pl.BlockSpec((_BS, H, W, D), lambda b: (b, 0, 0, 0)),
        ],
        out_specs=[
            pl.BlockSpec((_BS, H, W, D), lambda b: (b, 0, 0, 0)),
            pl.BlockSpec((_BS, 1, 128), lambda b: (b, 0, 0)),
        ],
        out_shape=[
            jax.ShapeDtypeStruct((B, H, W, D), jnp.float32),
            jax.ShapeDtypeStruct((B, 1, 128), jnp.float32),
        ],
        compiler_params=pltpu.CompilerParams(
            dimension_semantics=("parallel",),
        ),
    )(pf, xt, tt)

    selected = jnp.transpose(sel, (0, 3, 1, 2))
    min_loss = ml[:, 0, 0]
    return selected, min_loss


# speedup vs baseline: 1.5428x; 1.0307x over previous
import math

import jax
import jax.numpy as jnp
from jax.experimental import pallas as pl
from jax.experimental.pallas import tpu as pltpu

_K = 8
_BS = 2


def _body(pf_ref, x_ref, t_ref, sel_ref, ml_ref):
    _, h, w, d = t_ref.shape
    b = pl.program_id(0)
    inv_n = 1.0 / (h * w * d)
    penalty = math.log(_K, 2) / (h * w)

    for s in range(_BS):
        tb = t_ref[s]

        best_scaled = jnp.float32(jnp.inf)
        best_loss = jnp.float32(0.0)
        best_idx = jnp.int32(0)
        for k in range(_K):
            chunk = x_ref[s, :, :, k * d:(k + 1) * d]
            diff = chunk - tb
            loss_k = jnp.sum(diff * diff) * inv_n + penalty
            scaled_k = loss_k * pf_ref[0, k]
            better = scaled_k < best_scaled
            best_scaled = jnp.where(better, scaled_k, best_scaled)
            best_loss = jnp.where(better, loss_k, best_loss)
            best_idx = jnp.where(better, jnp.int32(k), best_idx)

        ml_ref[b * _BS + s] = best_loss
        for k in range(_K):
            @pl.when(best_idx == k)
            def _():
                sel_ref[s] = x_ref[s, :, :, k * d:(k + 1) * d]


def kernel(x, target, pick_frequency):
    B, C, H, W = x.shape
    D = C // _K
    xt = jnp.transpose(x, (0, 2, 3, 1))
    tt = jnp.transpose(target, (0, 2, 3, 1))
    pf = pick_frequency.reshape(1, _K)

    sel, ml = pl.pallas_call(
        _body,
        grid=(B // _BS,),
        in_specs=[
            pl.BlockSpec(memory_space=pltpu.SMEM),
            pl.BlockSpec((_BS, H, W, C), lambda b: (b, 0, 0, 0)),
            pl.BlockSpec((_BS, H, W, D), lambda b: (b, 0, 0, 0)),
        ],
        out_specs=[
            pl.BlockSpec((_BS, H, W, D), lambda b: (b, 0, 0, 0)),
            pl.BlockSpec(memory_space=pltpu.SMEM),
        ],
        out_shape=[
            jax.ShapeDtypeStruct((B, H, W, D), jnp.float32),
            jax.ShapeDtypeStruct((B,), jnp.float32),
        ],
        compiler_params=pltpu.CompilerParams(
            dimension_semantics=("parallel",),
        ),
    )(pf, xt, tt)

    selected = jnp.transpose(sel, (0, 3, 1, 2))
    return selected, ml
